# Initial kernel scaffold; baseline (speedup 1.0000x reference)
import jax
import jax.numpy as jnp
from jax.experimental import pallas as pl


def kernel(seq, table):
    raise NotImplementedError("write your pallas kernel here")


# speedup vs baseline: 2.5572x; 2.5572x over previous
import functools

import jax
import jax.numpy as jnp
from jax import lax
from jax.experimental import pallas as pl
from jax.experimental.pallas import tpu as pltpu
from jax.experimental.pallas import tpu_sc as plsc

B, L, V, D = 1024, 4096, 6, 128
VP = 8
LANES = 16
DC = 8
NDC = D // DC
NG = L // LANES

_info = plsc.get_sparse_core_info()
_NC, _NS = _info.num_cores, _info.num_subcores
NW = _NC * _NS
BPW = B // NW


@functools.partial(
    pl.kernel,
    mesh=plsc.VectorSubcoreMesh(core_axis_name="c", subcore_axis_name="s"),
    out_type=jax.ShapeDtypeStruct((B * D * L,), jnp.float32),
    compiler_params=pltpu.CompilerParams(needs_layout_passes=False),
    scratch_types=[
        pltpu.VMEM((D * VP,), jnp.float32),
        pltpu.VMEM((L,), jnp.int32),
        pltpu.VMEM((DC * L,), jnp.float32),
    ],
)
def _emb_lookup(seq_hbm, tab_hbm, out_hbm, tab_v, seq_v, out_v):
    wid = lax.axis_index("s") * _NC + lax.axis_index("c")
    pltpu.sync_copy(tab_hbm, tab_v)

    def per_b(bb, carry):
        b = wid * BPW + bb
        pltpu.sync_copy(seq_hbm.at[pl.ds(b * L, L)], seq_v)

        def per_dc(dc, carry2):
            tab_base = dc * (DC * VP)

            def per_g(g, carry3):
                s = seq_v[pl.ds(g * LANES, LANES)]
                for dd in range(DC):
                    idx = s + (tab_base + dd * VP)
                    vals = plsc.load_gather(tab_v, [idx])
                    out_v[pl.ds(dd * L + g * LANES, LANES)] = vals
                return carry3

            lax.fori_loop(0, NG, per_g, 0, unroll=2)
            off = b * (D * L) + dc * (DC * L)
            pltpu.sync_copy(out_v, out_hbm.at[pl.ds(off, DC * L)])
            return carry2

        lax.fori_loop(0, NDC, per_dc, 0)
        return carry

    lax.fori_loop(0, BPW, per_b, 0)


def kernel(seq, table):
    seq = seq.astype(jnp.int32)
    table_eff = table.at[0].set(0.0)
    tab_flat = jnp.pad(table_eff.T, ((0, 0), (0, VP - V))).reshape(-1)
    out = _emb_lookup(seq.reshape(-1), tab_flat)
    return out.reshape(B, D, L)


# speedup vs baseline: 6.3091x; 2.4672x over previous
import functools

import jax
import jax.numpy as jnp
from jax import lax
from jax.experimental import pallas as pl
from jax.experimental.pallas import tpu as pltpu
from jax.experimental.pallas import tpu_sc as plsc

B, L, V, D = 1024, 4096, 6, 128
VP = 8
LANES = 16
DC = 8
NDC = D // DC
NG = L // LANES
NBUF = 2

_info = plsc.get_sparse_core_info()
_NC, _NS = _info.num_cores, _info.num_subcores
NW = _NC * _NS
BPW = B // NW


@functools.partial(
    pl.kernel,
    mesh=plsc.VectorSubcoreMesh(core_axis_name="c", subcore_axis_name="s"),
    out_type=jax.ShapeDtypeStruct((B * D * L,), jnp.float32),
    compiler_params=pltpu.CompilerParams(needs_layout_passes=False),
    scratch_types=[
        pltpu.VMEM((D * VP,), jnp.float32),
        pltpu.VMEM((L,), jnp.int32),
        pltpu.VMEM((DC * L,), jnp.float32),
        pltpu.VMEM((DC * L,), jnp.float32),
        pltpu.SemaphoreType.DMA,
        pltpu.SemaphoreType.DMA,
    ],
)
def _emb_lookup(seq_hbm, tab_hbm, out_hbm, tab_v, seq_v, out_v0, out_v1,
                sem0, sem1):
    wid = lax.axis_index("s") * _NC + lax.axis_index("c")
    pltpu.sync_copy(tab_hbm, tab_v)
    bufs = (out_v0, out_v1)
    sems = (sem0, sem1)

    def fill(buf, dc):
        tab_base = dc * (DC * VP)

        @plsc.parallel_loop(0, NG, unroll=2)
        def per_g(g):
            s = seq_v[pl.ds(g * LANES, LANES)]
            for dd in range(DC):
                idx = s + (tab_base + dd * VP)
                vals = plsc.load_gather(tab_v, [idx])
                buf[pl.ds(dd * L + g * LANES, LANES)] = vals

    def per_b(bb, carry):
        b = wid * BPW + bb
        pltpu.sync_copy(seq_hbm.at[pl.ds(b * L, L)], seq_v)

        def per_dc2(dc2, carry2):
            for par in range(NBUF):
                dc = dc2 * NBUF + par

                @pl.when(dc2 > 0)
                def _wait_prev():
                    pltpu.make_async_copy(
                        bufs[par], out_hbm.at[pl.ds(0, DC * L)], sems[par]
                    ).wait()

                fill(bufs[par], dc)
                off = b * (D * L) + dc * (DC * L)
                pltpu.make_async_copy(
                    bufs[par], out_hbm.at[pl.ds(off, DC * L)], sems[par]
                ).start()
            return carry2

        lax.fori_loop(0, NDC // NBUF, per_dc2, 0)
        for par in range(NBUF):
            pltpu.make_async_copy(
                bufs[par], out_hbm.at[pl.ds(0, DC * L)], sems[par]
            ).wait()
        return carry

    lax.fori_loop(0, BPW, per_b, 0)


def kernel(seq, table):
    seq = seq.astype(jnp.int32)
    table_eff = table.at[0].set(0.0)
    tab_flat = jnp.pad(table_eff.T, ((0, 0), (0, VP - V))).reshape(-1)
    out = _emb_lookup(seq.reshape(-1), tab_flat)
    return out.reshape(B, D, L)


# speedup vs baseline: 23.3149x; 3.6954x over previous
import functools

import jax
import jax.numpy as jnp
from jax import lax
from jax.experimental import pallas as pl
from jax.experimental.pallas import tpu as pltpu
from jax.experimental.pallas import tpu_sc as plsc

B, L, V, D = 1024, 4096, 6, 128
VP = 8
LANES = 16
DC = 8
NDC = D // DC
NG = L // LANES
NBUF = 2

_info = plsc.get_sparse_core_info()
_NC, _NS = _info.num_cores, _info.num_subcores
NW = _NC * _NS
BPW = B // NW


@functools.partial(
    pl.kernel,
    mesh=plsc.VectorSubcoreMesh(core_axis_name="c", subcore_axis_name="s"),
    out_type=jax.ShapeDtypeStruct((B, D, L), jnp.float32),
    compiler_params=pltpu.CompilerParams(needs_layout_passes=False),
    scratch_types=[
        pltpu.VMEM((D * VP,), jnp.float32),
        pltpu.VMEM((L,), jnp.int32),
        pltpu.VMEM((DC, L), jnp.float32),
        pltpu.VMEM((DC, L), jnp.float32),
        pltpu.SemaphoreType.DMA,
        pltpu.SemaphoreType.DMA,
    ],
)
def _emb_lookup(seq_hbm, tab_hbm, out_hbm, tab_v, seq_v, out_v0, out_v1,
                sem0, sem1):
    wid = lax.axis_index("s") * _NC + lax.axis_index("c")
    pltpu.sync_copy(tab_hbm, tab_v)
    bufs = (out_v0, out_v1)
    sems = (sem0, sem1)

    def fill(buf, dc):
        tab_base = dc * (DC * VP)

        @plsc.parallel_loop(0, NG, unroll=2)
        def per_g(g):
            s = seq_v[pl.ds(g * LANES, LANES)]
            for dd in range(DC):
                idx = s + (tab_base + dd * VP)
                vals = plsc.load_gather(tab_v, [idx])
                buf[dd, pl.ds(g * LANES, LANES)] = vals

    def per_b(bb, carry):
        b = wid * BPW + bb
        pltpu.sync_copy(seq_hbm.at[pl.ds(b * L, L)], seq_v)

        def per_dc2(dc2, carry2):
            for par in range(NBUF):
                dc = dc2 * NBUF + par

                @pl.when(dc2 > 0)
                def _wait_prev():
                    pltpu.make_async_copy(
                        bufs[par], out_hbm.at[0, pl.ds(0, DC)], sems[par]
                    ).wait()

                fill(bufs[par], dc)
                pltpu.make_async_copy(
                    bufs[par], out_hbm.at[b, pl.ds(dc * DC, DC)], sems[par]
                ).start()
            return carry2

        lax.fori_loop(0, NDC // NBUF, per_dc2, 0)
        for par in range(NBUF):
            pltpu.make_async_copy(
                bufs[par], out_hbm.at[0, pl.ds(0, DC)], sems[par]
            ).wait()
        return carry

    lax.fori_loop(0, BPW, per_b, 0)


def kernel(seq, table):
    seq = seq.astype(jnp.int32)
    table_eff = table.at[0].set(0.0)
    tab_flat = jnp.pad(table_eff.T, ((0, 0), (0, VP - V))).reshape(-1)
    return _emb_lookup(seq.reshape(-1), tab_flat)
